# baseline (speedup 1.0000x reference)
import functools

import jax
import jax.numpy as jnp
from jax import lax
from jax.experimental import pallas as pl
from jax.experimental.pallas import tpu as pltpu
from jax.experimental.pallas import tpu_sc as plsc

_N = 320000
_D = 128
_C = 1000
_CPAD = 1024
_NC = 2
_NS = 16
_NW = _NC * _NS
_RPW = _N // _NW
_K = 80
_NB = 4
_CHUNKS = _RPW // _K
_LPS = _CPAD // _NS


def _sc_body(feat_hbm, lab_hbm, sums_hbm, cnts_hbm,
             acc, featbuf, labbuf, zbuf, cntv, tab,
             fsem, ssem):
    c = lax.axis_index("c")
    s = lax.axis_index("s")
    wid = s * _NC + c
    base = wid * _RPW

    zero = jnp.zeros((16,), jnp.float32)
    one = jnp.ones((16,), jnp.float32)

    def _zrow(i, _):
        for j in range(_D // 16):
            zbuf[i, pl.ds(j * 16, 16)] = zero
        return 0

    lax.fori_loop(0, _LPS, _zrow, 0)

    def _ztab(j, _):
        tab[j] = 0
        return 0

    lax.fori_loop(0, _CPAD, _ztab, 0)

    pltpu.sync_copy(lab_hbm.at[wid], labbuf)
    pltpu.sync_copy(zbuf, acc.at[pl.ds(s * _LPS, _LPS)])
    plsc.subcore_barrier()

    def _feat_dma(t, p):
        return pltpu.make_async_copy(
            feat_hbm.at[pl.ds(base + t * _K, _K)],
            featbuf.at[pl.ds(p * _K, _K)], fsem.at[p])

    def _scatters(t, p):
        return pltpu.async_copy(featbuf.at[pl.ds(p * _K, _K)],
                                acc.at[labbuf.at[t]], ssem.at[p], add=True)

    def _wait_scatters(t, p):
        pltpu.make_async_copy(featbuf.at[pl.ds(p * _K, _K)],
                              acc.at[labbuf.at[t]], ssem.at[p]).wait()

    for t0 in range(_NB - 1):
        _feat_dma(t0, t0).start()

    def _chunk(t, accs):
        p = lax.rem(t, _NB)

        @pl.when(t > 0)
        def _():
            _wait_scatters(t - 1, lax.rem(t - 1, _NB))

        @pl.when(t + _NB - 1 < _CHUNKS)
        def _():
            _feat_dma(t + _NB - 1, lax.rem(t + _NB - 1, _NB)).start()

        _feat_dma(t, p).wait()
        _scatters(t, p)
        return accs

        for j in range(_K // 16):
            v = labbuf[t, pl.ds(j * 16, 16)]
            first = v[0]
            same = first == v[15]

            @pl.when(same)
            def _():
                tab[first] = tab[first] + 16

            @pl.when(jnp.logical_not(same))
            def _():
                for l in range(16):
                    idx = v[l]
                    tab[idx] = tab[idx] + 1

    lax.fori_loop(0, _CHUNKS, _chunk, 0)
    _wait_scatters(_CHUNKS - 1, lax.rem(_CHUNKS - 1, 2))

    lanes = lax.iota(jnp.int32, 16)

    def _cvt(j, _):
        vec = jnp.zeros((16,), jnp.float32)
        for l in range(16):
            sval = (tab[j * 16 + l]).astype(jnp.float32)
            vec = jnp.where(lanes == l, jnp.full((16,), sval, jnp.float32), vec)
        cntv[pl.ds(j * 16, 16)] = vec
        return 0

    lax.fori_loop(0, _CPAD // 16, _cvt, 0)
    pltpu.sync_copy(cntv, cnts_hbm.at[wid])

    plsc.subcore_barrier()
    pltpu.sync_copy(acc.at[pl.ds(s * _LPS, _LPS)],
                    sums_hbm.at[c, pl.ds(s * _LPS, _LPS)])


_sc_pass = pl.kernel(
    _sc_body,
    out_type=(
        jax.ShapeDtypeStruct((_NC, _CPAD, _D), jnp.float32),
        jax.ShapeDtypeStruct((_NW, _CPAD), jnp.float32),
    ),
    mesh=plsc.VectorSubcoreMesh(core_axis_name="c", subcore_axis_name="s"),
    scratch_types=[
        pltpu.VMEM_SHARED((_CPAD, _D), jnp.float32),
        pltpu.VMEM((_NB * _K, _D), jnp.float32),
        pltpu.VMEM((_CHUNKS, _K), jnp.int32),
        pltpu.VMEM((_LPS, _D), jnp.float32),
        pltpu.VMEM((_CPAD,), jnp.float32),
        pltpu.SMEM((_CPAD,), jnp.int32),
        pltpu.SemaphoreType.DMA((_NB,)),
        pltpu.SemaphoreType.DMA((_NB,)),
    ],
    name="wb_sc_pass",
)


_SSQ_BLK = 2000


def _ssq_body(x_ref, o_ref):
    i = pl.program_id(0)

    @pl.when(i == 0)
    def _():
        o_ref[...] = jnp.zeros((1, 1), jnp.float32)

    x = x_ref[...]
    o_ref[...] = o_ref[...] + jnp.full((1, 1), jnp.sum(x * x), jnp.float32)


_tc_ssq = pl.pallas_call(
    _ssq_body,
    grid=(_N // _SSQ_BLK,),
    in_specs=[pl.BlockSpec((_SSQ_BLK, _D), lambda i: (i, 0))],
    out_specs=pl.BlockSpec((1, 1), lambda i: (0, 0)),
    out_shape=jax.ShapeDtypeStruct((1, 1), jnp.float32),
)


def _combine_body(sums_ref, cnts_ref, ssqs_ref, out_ref):
    S = sums_ref[0] + sums_ref[1]
    n = jnp.sum(cnts_ref[...], axis=0)
    q_c = jnp.sum(S * S, axis=1)
    Q = jnp.sum(jnp.where(n > 0.0, q_c / jnp.maximum(n, 1.0), 0.0))
    T = jnp.sum(S, axis=0)
    ssq = ssqs_ref[0, 0]
    W = ssq - Q
    B = Q - jnp.sum(T * T) / _N
    out_ref[...] = jnp.full((1, 1), W / (B + 1e-8), jnp.float32)


_combine = pl.pallas_call(
    _combine_body,
    out_shape=jax.ShapeDtypeStruct((1, 1), jnp.float32),
)


@jax.jit
def kernel(features, labels):
    labels = labels.astype(jnp.int32).reshape(_NW, _CHUNKS, _K)
    sums, cnts = _sc_pass(features, labels)
    ssq = _tc_ssq(features)
    return _combine(sums, cnts, ssq)[0, 0]


# speedup vs baseline: 1.3466x; 1.3466x over previous
import functools

import jax
import jax.numpy as jnp
from jax import lax
from jax.experimental import pallas as pl
from jax.experimental.pallas import tpu as pltpu
from jax.experimental.pallas import tpu_sc as plsc

_N = 320000
_D = 128
_C = 1000
_CPAD = 1024
_NC = 2
_NS = 16
_NW = _NC * _NS
_RPW = _N // _NW
_K = 80
_NB = 4
_CHUNKS = _RPW // _K
_LPS = _CPAD // _NS


def _sc_body(feat_hbm, lab_hbm, sums_hbm, cnts_hbm, ssqs_hbm,
             acc, featbuf, labbuf, zbuf, cntv, ssqbuf, tab,
             fsem, ssem):
    c = lax.axis_index("c")
    s = lax.axis_index("s")
    wid = s * _NC + c
    base = wid * _RPW

    zero = jnp.zeros((16,), jnp.float32)
    one = jnp.ones((16,), jnp.float32)

    def _zrow(i, _):
        for j in range(_D // 16):
            zbuf[i, pl.ds(j * 16, 16)] = zero
        return 0

    lax.fori_loop(0, _LPS, _zrow, 0)

    def _ztab(j, _):
        tab[j] = 0
        return 0

    lax.fori_loop(0, _CPAD, _ztab, 0)

    pltpu.sync_copy(lab_hbm.at[wid], labbuf)
    pltpu.sync_copy(zbuf, acc.at[pl.ds(s * _LPS, _LPS)])
    plsc.subcore_barrier()

    def _feat_dma(t, p):
        return pltpu.make_async_copy(
            feat_hbm.at[pl.ds(base + t * _K, _K)],
            featbuf.at[pl.ds(p * _K, _K)], fsem.at[p])

    def _scatters(t, p):
        return pltpu.async_copy(featbuf.at[pl.ds(p * _K, _K)],
                                acc.at[labbuf.at[t]], ssem.at[p], add=True)

    def _wait_scatters(t, p):
        pltpu.make_async_copy(featbuf.at[pl.ds(p * _K, _K)],
                              acc.at[labbuf.at[t]], ssem.at[p]).wait()

    for t0 in range(_NB - 1):
        _feat_dma(t0, t0).start()

    def _chunk(t, accs):
        p = lax.rem(t, _NB)

        @pl.when(t > 0)
        def _():
            _wait_scatters(t - 1, lax.rem(t - 1, _NB))

        @pl.when(t + _NB - 1 < _CHUNKS)
        def _():
            _feat_dma(t + _NB - 1, lax.rem(t + _NB - 1, _NB)).start()

        _feat_dma(t, p).wait()
        _scatters(t, p)

        for j in range(_K // 16):
            v = labbuf[t, pl.ds(j * 16, 16)]
            first = v[0]
            same = first == v[15]

            @pl.when(same)
            def _():
                tab[first] = tab[first] + 16

            @pl.when(jnp.logical_not(same))
            def _():
                for l in range(16):
                    idx = v[l]
                    tab[idx] = tab[idx] + 1

        def _rows(i, a):
            out = list(a)
            for r in range(4):
                for j in range(_D // 16):
                    v = featbuf[p * _K + i * 4 + r, pl.ds(j * 16, 16)]
                    out[j] = out[j] + v * v
            return tuple(out)

        return lax.fori_loop(0, _K // 4, _rows, accs)

    accs = lax.fori_loop(0, _CHUNKS, _chunk,
                         tuple(zero for _ in range(_D // 16)))
    _wait_scatters(_CHUNKS - 1, lax.rem(_CHUNKS - 1, 2))

    tot = accs[0]
    for j in range(1, _D // 16):
        tot = tot + accs[j]
    ssqbuf[pl.ds(0, 16)] = tot
    pltpu.sync_copy(ssqbuf, ssqs_hbm.at[wid])

    lanes = lax.iota(jnp.int32, 16)

    def _cvt(j, _):
        vec = jnp.zeros((16,), jnp.float32)
        for l in range(16):
            sval = (tab[j * 16 + l]).astype(jnp.float32)
            vec = jnp.where(lanes == l, jnp.full((16,), sval, jnp.float32), vec)
        cntv[pl.ds(j * 16, 16)] = vec
        return 0

    lax.fori_loop(0, _CPAD // 16, _cvt, 0)
    pltpu.sync_copy(cntv, cnts_hbm.at[wid])

    plsc.subcore_barrier()
    pltpu.sync_copy(acc.at[pl.ds(s * _LPS, _LPS)],
                    sums_hbm.at[c, pl.ds(s * _LPS, _LPS)])


_sc_pass = pl.kernel(
    _sc_body,
    out_type=(
        jax.ShapeDtypeStruct((_NC, _CPAD, _D), jnp.float32),
        jax.ShapeDtypeStruct((_NW, _CPAD), jnp.float32),
        jax.ShapeDtypeStruct((_NW, 16), jnp.float32),
    ),
    mesh=plsc.VectorSubcoreMesh(core_axis_name="c", subcore_axis_name="s"),
    scratch_types=[
        pltpu.VMEM_SHARED((_CPAD, _D), jnp.float32),
        pltpu.VMEM((_NB * _K, _D), jnp.float32),
        pltpu.VMEM((_CHUNKS, _K), jnp.int32),
        pltpu.VMEM((_LPS, _D), jnp.float32),
        pltpu.VMEM((_CPAD,), jnp.float32),
        pltpu.VMEM((16,), jnp.float32),
        pltpu.SMEM((_CPAD,), jnp.int32),
        pltpu.SemaphoreType.DMA((_NB,)),
        pltpu.SemaphoreType.DMA((_NB,)),
    ],
    name="wb_sc_pass",
)


def _combine_body(sums_ref, cnts_ref, ssqs_ref, out_ref):
    S = sums_ref[0] + sums_ref[1]
    n = jnp.sum(cnts_ref[...], axis=0)
    q_c = jnp.sum(S * S, axis=1)
    Q = jnp.sum(jnp.where(n > 0.0, q_c / jnp.maximum(n, 1.0), 0.0))
    T = jnp.sum(S, axis=0)
    ssq = jnp.sum(ssqs_ref[...])
    W = ssq - Q
    B = Q - jnp.sum(T * T) / _N
    out_ref[...] = jnp.full((1, 1), W / (B + 1e-8), jnp.float32)


_combine = pl.pallas_call(
    _combine_body,
    out_shape=jax.ShapeDtypeStruct((1, 1), jnp.float32),
)


@jax.jit
def kernel(features, labels):
    labels = labels.astype(jnp.int32).reshape(_NW, _CHUNKS, _K)
    sums, cnts, ssqs = _sc_pass(features, labels)
    return _combine(sums, cnts, ssqs)[0, 0]
